# baseline (speedup 1.0000x reference)
import functools

import jax
import jax.numpy as jnp
from jax import lax
from jax.experimental import pallas as pl
from jax.experimental.pallas import tpu as pltpu
from jax.experimental.pallas import tpu_sc as plsc

ITEM_N = 100000
USER_N = 16384
EMB = 32
NSUB = 16
B = 128


def _make_conv(nr_out: int):
  r_sc = nr_out // 2
  wz = ((r_sc // NSUB) + 7) // 8 * 8
  mesh = plsc.VectorSubcoreMesh(core_axis_name="c", subcore_axis_name="s")

  def body(x_hbm, cols_hbm, rows_hbm, vals_hbm, bounds_hbm, out_hbm,
           acc_sh, bounds_v, cols0, cols1, rows0, rows1, vals0, vals1,
           srow0, srow1, buf0, buf1, sem_g0, sem_g1, sem_i0, sem_i1,
           sem_s0, sem_s1):
    c = lax.axis_index("c")
    s = lax.axis_index("s")
    zero16 = jnp.zeros((16,), jnp.float32)

    for r in range(B):
      for h in range(2):
        buf0[r, pl.ds(h * 16, 16)] = zero16
    s0 = pl.multiple_of(jnp.minimum(s * wz, r_sc - wz), 8)
    done = 0
    while done < wz:
      n = min(B, wz - done)
      pltpu.sync_copy(buf0.at[pl.ds(0, n)], acc_sh.at[pl.ds(s0 + done, n)])
      done += n

    pltpu.sync_copy(bounds_hbm, bounds_v)
    b16 = bounds_v[...]
    e0 = jnp.where(c == 0, b16[0], b16[1])
    e1 = jnp.where(c == 0, b16[1], b16[2])
    per = (e1 - e0 + NSUB - 1) // NSUB
    start = e0 + s * per
    end = jnp.minimum(start + per, e1)
    start = jnp.minimum(start, end)
    astart = pl.multiple_of((start // 8) * 8, 8)
    nch = (end - astart + B - 1) // B
    row_base = pl.multiple_of(c * r_sc, 8)

    def chunk_off(k):
      return pl.multiple_of(astart + k * B, 8)

    def start_idx(k, cv, rv, vv, sem):
      off = chunk_off(k)
      pltpu.async_copy(cols_hbm.at[pl.ds(off, B)], cv, sem)
      pltpu.async_copy(rows_hbm.at[pl.ds(off, B)], rv, sem)
      pltpu.async_copy(vals_hbm.at[pl.ds(off, B)], vv, sem)

    def wait_idx(cv, rv, vv, sem):
      pltpu.make_async_copy(cols_hbm.at[pl.ds(0, B)], cv, sem).wait()
      pltpu.make_async_copy(rows_hbm.at[pl.ds(0, B)], rv, sem).wait()
      pltpu.make_async_copy(vals_hbm.at[pl.ds(0, B)], vv, sem).wait()

    def start_gather(cv, bf, sem):
      pltpu.async_copy(x_hbm.at[cv], bf, sem)

    def wait_gather(cv, bf, sem):
      pltpu.make_async_copy(x_hbm.at[cv], bf, sem).wait()

    def process(k, rv, vv, bf, srow, sem_s):
      off = chunk_off(k)
      for g in range(B // 16):
        sl = pl.ds(g * 16, 16)
        r16 = rv[sl] - row_base
        r16 = jnp.minimum(jnp.maximum(r16, 0), r_sc - 1)
        srow[sl] = r16
      for g in range(B // 16):
        vg = vv[pl.ds(g * 16, 16)]
        for l in range(16):
          r = g * 16 + l
          e = off + r
          v = jnp.where((e >= start) & (e < end), vg[l], 0.0)
          for h in range(2):
            hs = pl.ds(h * 16, 16)
            bf[r, hs] = bf[r, hs] * v
      pltpu.async_copy(bf, acc_sh.at[srow], sem_s, add=True)

    def wait_scatter(bf, srow, sem_s):
      pltpu.make_async_copy(bf, acc_sh.at[srow], sem_s).wait()

    plsc.subcore_barrier()

    @pl.when(nch >= 1)
    def _prologue():
      start_idx(0, cols0, rows0, vals0, sem_i0)
      wait_idx(cols0, rows0, vals0, sem_i0)
      start_gather(cols0, buf0, sem_g0)

      @pl.when(nch >= 2)
      def _():
        start_idx(1, cols1, rows1, vals1, sem_i1)

    def pair(j, carry):
      k0 = 2 * j
      k1 = k0 + 1

      @pl.when(k1 < nch)
      def _():
        @pl.when(j >= 1)
        def _():
          wait_scatter(buf1, srow1, sem_s1)

        wait_idx(cols1, rows1, vals1, sem_i1)
        start_gather(cols1, buf1, sem_g1)

      wait_gather(cols0, buf0, sem_g0)
      process(k0, rows0, vals0, buf0, srow0, sem_s0)

      @pl.when(k0 + 2 < nch)
      def _():
        start_idx(k0 + 2, cols0, rows0, vals0, sem_i0)

      @pl.when(k1 < nch)
      def _():
        wait_gather(cols1, buf1, sem_g1)

        @pl.when(k1 + 2 < nch)
        def _():
          start_idx(k1 + 2, cols1, rows1, vals1, sem_i1)

        process(k1, rows1, vals1, buf1, srow1, sem_s1)

      @pl.when(k0 + 2 < nch)
      def _():
        wait_scatter(buf0, srow0, sem_s0)
        wait_idx(cols0, rows0, vals0, sem_i0)
        start_gather(cols0, buf0, sem_g0)

      return carry

    lax.fori_loop(0, (nch + 1) // 2, pair, 0)

    @pl.when(nch >= 1)
    def _():
      wait_scatter(buf0, srow0, sem_s0)

    @pl.when(nch >= 2)
    def _():
      wait_scatter(buf1, srow1, sem_s1)

    plsc.subcore_barrier()
    pltpu.sync_copy(acc_sh.at[pl.ds(s0, wz)],
                    out_hbm.at[pl.ds(row_base + s0, wz)])

  return pl.kernel(
      body,
      out_type=jax.ShapeDtypeStruct((nr_out, EMB), jnp.float32),
      mesh=mesh,
      scratch_types=[
          pltpu.VMEM_SHARED((r_sc, EMB), jnp.float32),
          pltpu.VMEM((16,), jnp.int32),
          pltpu.VMEM((B,), jnp.int32),
          pltpu.VMEM((B,), jnp.int32),
          pltpu.VMEM((B,), jnp.int32),
          pltpu.VMEM((B,), jnp.int32),
          pltpu.VMEM((B,), jnp.float32),
          pltpu.VMEM((B,), jnp.float32),
          pltpu.VMEM((B,), jnp.int32),
          pltpu.VMEM((B,), jnp.int32),
          pltpu.VMEM((B, EMB), jnp.float32),
          pltpu.VMEM((B, EMB), jnp.float32),
          pltpu.SemaphoreType.DMA,
          pltpu.SemaphoreType.DMA,
          pltpu.SemaphoreType.DMA,
          pltpu.SemaphoreType.DMA,
          pltpu.SemaphoreType.DMA,
          pltpu.SemaphoreType.DMA,
      ],
      compiler_params=pltpu.CompilerParams(use_tc_tiling_on_sc=False),
  )


def _prep(rows, cols, vals, nr_out, x_rows):
  nnz = rows.shape[0]
  split = jnp.searchsorted(rows, nr_out // 2, side="left").astype(jnp.int32)
  bounds = jnp.zeros((16,), jnp.int32).at[1].set(split).at[2].set(nnz)
  padn = 4 * B
  pad_cols = (jnp.arange(padn, dtype=jnp.int32) * 997) % x_rows
  cols_p = jnp.concatenate([cols, pad_cols])
  rows_p = jnp.concatenate([rows, jnp.zeros((padn,), jnp.int32)])
  vals_p = jnp.concatenate([vals, jnp.zeros((padn,), jnp.float32)])
  return cols_p, rows_p, vals_p, bounds


@jax.jit
def kernel(item_embeddings, s_vals, u_vals, s_rows, s_cols, u_rows, u_cols):
  conv_i = _make_conv(ITEM_N)
  conv_u = _make_conv(USER_N)
  sc, sr, sv, sb = _prep(s_rows, s_cols, s_vals, ITEM_N, ITEM_N)
  uc, ur, uv, ub = _prep(u_rows, u_cols, u_vals, USER_N, ITEM_N)
  x1 = conv_i(item_embeddings, sc, sr, sv, sb)
  x2 = conv_i(x1, sc, sr, sv, sb)
  user = conv_u(x2, uc, ur, uv, ub)
  return (user, x2)


# speedup vs baseline: 1.0042x; 1.0042x over previous
import functools

import jax
import jax.numpy as jnp
from jax import lax
from jax.experimental import pallas as pl
from jax.experimental.pallas import tpu as pltpu
from jax.experimental.pallas import tpu_sc as plsc

ITEM_N = 100000
USER_N = 16384
EMB = 32
NSUB = 16
B = 128


def _make_conv(nr_out: int):
  r_sc = nr_out // 2
  wz = ((r_sc // NSUB) + 7) // 8 * 8
  mesh = plsc.VectorSubcoreMesh(core_axis_name="c", subcore_axis_name="s")

  def body(x_hbm, cols_hbm, rows_hbm, vals_hbm, bounds_hbm, out_hbm,
           acc_sh, bounds_v, cols0, cols1, rows0, rows1, vals0, vals1,
           srow0, srow1, buf0, buf1, sem_g0, sem_g1, sem_i0, sem_i1,
           sem_s0, sem_s1):
    c = lax.axis_index("c")
    s = lax.axis_index("s")
    zero16 = jnp.zeros((16,), jnp.float32)

    for r in range(B):
      for h in range(2):
        buf0[r, pl.ds(h * 16, 16)] = zero16
    s0 = pl.multiple_of(jnp.minimum(s * wz, r_sc - wz), 8)
    done = 0
    while done < wz:
      n = min(B, wz - done)
      pltpu.sync_copy(buf0.at[pl.ds(0, n)], acc_sh.at[pl.ds(s0 + done, n)])
      done += n

    pltpu.sync_copy(bounds_hbm, bounds_v)
    b16 = bounds_v[...]
    e0 = jnp.where(c == 0, b16[0], b16[1])
    e1 = jnp.where(c == 0, b16[1], b16[2])
    per = (e1 - e0 + NSUB - 1) // NSUB
    start = e0 + s * per
    end = jnp.minimum(start + per, e1)
    start = jnp.minimum(start, end)
    astart = pl.multiple_of((start // 8) * 8, 8)
    nch = (end - astart + B - 1) // B
    row_base = pl.multiple_of(c * r_sc, 8)

    def chunk_off(k):
      return pl.multiple_of(astart + k * B, 8)

    def start_idx(k, cv, rv, vv, sem):
      off = chunk_off(k)
      pltpu.async_copy(cols_hbm.at[pl.ds(off, B)], cv, sem)
      pltpu.async_copy(rows_hbm.at[pl.ds(off, B)], rv, sem)
      pltpu.async_copy(vals_hbm.at[pl.ds(off, B)], vv, sem)

    def wait_idx(cv, rv, vv, sem):
      pltpu.make_async_copy(cols_hbm.at[pl.ds(0, B)], cv, sem).wait()
      pltpu.make_async_copy(rows_hbm.at[pl.ds(0, B)], rv, sem).wait()
      pltpu.make_async_copy(vals_hbm.at[pl.ds(0, B)], vv, sem).wait()

    def start_gather(cv, bf, sem):
      pltpu.async_copy(x_hbm.at[cv], bf, sem)

    def wait_gather(cv, bf, sem):
      pltpu.make_async_copy(x_hbm.at[cv], bf, sem).wait()

    def process(k, rv, vv, bf, srow, sem_s):
      off = chunk_off(k)
      for g in range(B // 16):
        sl = pl.ds(g * 16, 16)
        r16 = rv[sl] - row_base
        r16 = jnp.minimum(jnp.maximum(r16, 0), r_sc - 1)
        srow[sl] = r16
      iota16 = lax.broadcasted_iota(jnp.int32, (16,), 0)
      for g in range(B // 16):
        vg = vv[pl.ds(g * 16, 16)]
        e16 = (off + g * 16) + iota16
        vm = jnp.where((e16 >= start) & (e16 < end), vg, 0.0)
        for l in range(16):
          r = g * 16 + l
          v = vm[l]
          for h in range(2):
            hs = pl.ds(h * 16, 16)
            bf[r, hs] = bf[r, hs] * v
      pltpu.async_copy(bf, acc_sh.at[srow], sem_s, add=True)

    def wait_scatter(bf, srow, sem_s):
      pltpu.make_async_copy(bf, acc_sh.at[srow], sem_s).wait()

    plsc.subcore_barrier()

    @pl.when(nch >= 1)
    def _prologue():
      start_idx(0, cols0, rows0, vals0, sem_i0)
      wait_idx(cols0, rows0, vals0, sem_i0)
      start_gather(cols0, buf0, sem_g0)

      @pl.when(nch >= 2)
      def _():
        start_idx(1, cols1, rows1, vals1, sem_i1)

    def pair(j, carry):
      k0 = 2 * j
      k1 = k0 + 1

      @pl.when(k1 < nch)
      def _():
        @pl.when(j >= 1)
        def _():
          wait_scatter(buf1, srow1, sem_s1)

        wait_idx(cols1, rows1, vals1, sem_i1)
        start_gather(cols1, buf1, sem_g1)

      wait_gather(cols0, buf0, sem_g0)
      process(k0, rows0, vals0, buf0, srow0, sem_s0)

      @pl.when(k0 + 2 < nch)
      def _():
        start_idx(k0 + 2, cols0, rows0, vals0, sem_i0)

      @pl.when(k1 < nch)
      def _():
        wait_gather(cols1, buf1, sem_g1)

        @pl.when(k1 + 2 < nch)
        def _():
          start_idx(k1 + 2, cols1, rows1, vals1, sem_i1)

        process(k1, rows1, vals1, buf1, srow1, sem_s1)

      @pl.when(k0 + 2 < nch)
      def _():
        wait_scatter(buf0, srow0, sem_s0)
        wait_idx(cols0, rows0, vals0, sem_i0)
        start_gather(cols0, buf0, sem_g0)

      return carry

    lax.fori_loop(0, (nch + 1) // 2, pair, 0)

    @pl.when(nch >= 1)
    def _():
      wait_scatter(buf0, srow0, sem_s0)

    @pl.when(nch >= 2)
    def _():
      wait_scatter(buf1, srow1, sem_s1)

    plsc.subcore_barrier()
    pltpu.sync_copy(acc_sh.at[pl.ds(s0, wz)],
                    out_hbm.at[pl.ds(row_base + s0, wz)])

  return pl.kernel(
      body,
      out_type=jax.ShapeDtypeStruct((nr_out, EMB), jnp.float32),
      mesh=mesh,
      scratch_types=[
          pltpu.VMEM_SHARED((r_sc, EMB), jnp.float32),
          pltpu.VMEM((16,), jnp.int32),
          pltpu.VMEM((B,), jnp.int32),
          pltpu.VMEM((B,), jnp.int32),
          pltpu.VMEM((B,), jnp.int32),
          pltpu.VMEM((B,), jnp.int32),
          pltpu.VMEM((B,), jnp.float32),
          pltpu.VMEM((B,), jnp.float32),
          pltpu.VMEM((B,), jnp.int32),
          pltpu.VMEM((B,), jnp.int32),
          pltpu.VMEM((B, EMB), jnp.float32),
          pltpu.VMEM((B, EMB), jnp.float32),
          pltpu.SemaphoreType.DMA,
          pltpu.SemaphoreType.DMA,
          pltpu.SemaphoreType.DMA,
          pltpu.SemaphoreType.DMA,
          pltpu.SemaphoreType.DMA,
          pltpu.SemaphoreType.DMA,
      ],
      compiler_params=pltpu.CompilerParams(use_tc_tiling_on_sc=False),
  )


def _prep(rows, cols, vals, nr_out, x_rows):
  nnz = rows.shape[0]
  split = jnp.searchsorted(rows, nr_out // 2, side="left").astype(jnp.int32)
  bounds = jnp.zeros((16,), jnp.int32).at[1].set(split).at[2].set(nnz)
  padn = 4 * B
  pad_cols = (jnp.arange(padn, dtype=jnp.int32) * 997) % x_rows
  cols_p = jnp.concatenate([cols, pad_cols])
  rows_p = jnp.concatenate([rows, jnp.zeros((padn,), jnp.int32)])
  vals_p = jnp.concatenate([vals, jnp.zeros((padn,), jnp.float32)])
  return cols_p, rows_p, vals_p, bounds


@jax.jit
def kernel(item_embeddings, s_vals, u_vals, s_rows, s_cols, u_rows, u_cols):
  conv_i = _make_conv(ITEM_N)
  conv_u = _make_conv(USER_N)
  sc, sr, sv, sb = _prep(s_rows, s_cols, s_vals, ITEM_N, ITEM_N)
  uc, ur, uv, ub = _prep(u_rows, u_cols, u_vals, USER_N, ITEM_N)
  x1 = conv_i(item_embeddings, sc, sr, sv, sb)
  x2 = conv_i(x1, sc, sr, sv, sb)
  user = conv_u(x2, uc, ur, uv, ub)
  return (user, x2)
